# baseline (speedup 1.0000x reference)
import functools

import jax
import jax.numpy as jnp
from jax import lax
from jax.experimental import pallas as pl
from jax.experimental.pallas import tpu as pltpu
from jax.experimental.pallas import tpu_sc as plsc


def _enc_vq_body(x_ref, w1_ref, b1_ref, w2_ref, b2_ref, cb_ref,
                 ze_ref, idx_ref, fidx_ref, loss_ref,
                 cb2_ref, cbn_ref,
                 *, G, K, GD):
    @pl.when(pl.program_id(0) == 0)
    def _prep():
        cb = cb_ref[...]
        cb2_ref[...] = cb * 2.0
        cbn_ref[...] = jnp.sum(cb * cb, axis=1)

    x = x_ref[...]
    h = jnp.maximum(jnp.dot(x, w1_ref[...]) + b1_ref[...], 0.0)
    z_e = jnp.dot(h, w2_ref[...]) + b2_ref[...]
    ze_ref[...] = z_e

    R = x.shape[0]
    iota_f = jax.lax.broadcasted_iota(jnp.int32, (R, K), 1).astype(jnp.float32)
    big = float(K)

    idx_parts = []
    block_sq = jnp.zeros((), jnp.float32)
    for g in range(G):
        zg = z_e[:, g * GD:(g + 1) * GD]
        s2 = jax.lax.dot_general(zg, cb2_ref[g], (((1,), (0,)), ((), ())))
        a = s2 - cbn_ref[g][None, :]
        m = jnp.max(a, axis=1, keepdims=True)
        key = jnp.where(a == m, iota_f, big)
        idxg = jnp.min(key, axis=1, keepdims=True)
        idx_parts.append(idxg)
        zn = jnp.sum(zg * zg, axis=1, keepdims=True)
        block_sq = block_sq + jnp.sum(zn - m)

    idx_f = jnp.concatenate(idx_parts, axis=1)
    idx = idx_f.astype(jnp.int32)
    idx_ref[...] = idx
    off = jax.lax.broadcasted_iota(jnp.int32, (R, G), 1) * K
    fidx_ref[...] = idx + off

    @pl.when(pl.program_id(0) == 0)
    def _init():
        loss_ref[...] = jnp.zeros((1, 1), jnp.float32)

    loss_ref[...] += block_sq[None, None]


def _dec_body(zq_ref, dw1_ref, db1_ref, dw2_ref, db2_ref, bh_ref):
    hd = jnp.maximum(jnp.dot(zq_ref[...], dw1_ref[...]) + db1_ref[...], 0.0)
    bh_ref[...] = jnp.dot(hd, dw2_ref[...]) + db2_ref[...]


def _sc_gather(table, fidx_flat, NB, GD):
    info = plsc.get_sparse_core_info()
    NW = info.num_cores * info.num_subcores
    CH = 128
    per_w = NB // NW
    n_chunks = per_w // CH
    mesh = plsc.VectorSubcoreMesh(core_axis_name="c", subcore_axis_name="s")

    @functools.partial(
        pl.kernel, mesh=mesh,
        compiler_params=pltpu.CompilerParams(use_tc_tiling_on_sc=False),
        out_type=jax.ShapeDtypeStruct((NB, GD), jnp.float32),
        scratch_types=[
            pltpu.VMEM((CH,), jnp.int32),
            pltpu.VMEM((CH, GD), jnp.float32),
            pltpu.SemaphoreType.DMA,
        ],
    )
    def k(table_hbm, idx_hbm, out_hbm, idx_v, rows_v, sem):
        wid = lax.axis_index("s") * info.num_cores + lax.axis_index("c")
        base = wid * per_w

        def body(j, carry):
            start = base + j * CH
            pltpu.sync_copy(idx_hbm.at[pl.ds(start, CH)], idx_v)
            pltpu.async_copy(table_hbm.at[idx_v], rows_v, sem).wait()
            pltpu.sync_copy(rows_v, out_hbm.at[pl.ds(start, CH)])
            return carry

        lax.fori_loop(0, n_chunks, body, 0)

    return k(table, fidx_flat)


def kernel(bands, enc_w1, enc_b1, enc_w2, enc_b2, codebooks,
           dec_w1, dec_b1, dec_w2, dec_b2):
    B, T, D = bands.shape
    N = B * T
    G, K, GD = codebooks.shape
    H = enc_w1.shape[1]
    L = enc_w2.shape[1]
    beta = 0.25

    R = min(2048, N)
    assert N % R == 0
    grid = (N // R,)

    x = bands.reshape(N, D)
    b1 = enc_b1.reshape(1, H)
    b2 = enc_b2.reshape(1, L)
    db1 = dec_b1.reshape(1, H)
    db2 = dec_b2.reshape(1, D)
    cb_t = codebooks.transpose(0, 2, 1)
    table = codebooks.reshape(G * K, GD)

    row_spec = lambda c: pl.BlockSpec((R, c), lambda i: (i, 0))
    full2 = lambda a, b: pl.BlockSpec((a, b), lambda i: (0, 0))

    def enc_vq(xc):
        n = xc.shape[0]
        return pl.pallas_call(
            functools.partial(_enc_vq_body, G=G, K=K, GD=GD),
            grid=(n // R,),
            in_specs=[
                row_spec(D),
                full2(D, H), full2(1, H), full2(H, L), full2(1, L),
                pl.BlockSpec((G, GD, K), lambda i: (0, 0, 0)),
            ],
            out_specs=(
                row_spec(L),
                pl.BlockSpec((R, G), lambda i: (i, 0)),
                pl.BlockSpec((R, G), lambda i: (i, 0)),
                pl.BlockSpec((1, 1), lambda i: (0, 0)),
            ),
            out_shape=(
                jax.ShapeDtypeStruct((n, L), jnp.float32),
                jax.ShapeDtypeStruct((n, G), jnp.int32),
                jax.ShapeDtypeStruct((n, G), jnp.int32),
                jax.ShapeDtypeStruct((1, 1), jnp.float32),
            ),
            scratch_shapes=[
                pltpu.VMEM((G, GD, K), jnp.float32),
                pltpu.VMEM((G, K), jnp.float32),
            ],
        )(xc, enc_w1, b1, enc_w2, b2, cb_t)

    def dec(zqc):
        n = zqc.shape[0]
        return pl.pallas_call(
            _dec_body,
            grid=(n // R,),
            in_specs=[
                row_spec(L),
                full2(L, H), full2(1, H), full2(H, D), full2(1, D),
            ],
            out_specs=row_spec(D),
            out_shape=jax.ShapeDtypeStruct((n, D), jnp.float32),
        )(zqc, dec_w1, db1, dec_w2, db2)

    halves = 2 if N % (2 * R) == 0 and (N // 2) * G % (32 * 128) == 0 else 1
    NH = N // halves
    parts = []
    for hh in range(halves):
        xc = jax.lax.slice_in_dim(x, hh * NH, (hh + 1) * NH, axis=0)
        z_e_c, idx_c, fidx_c, loss_c = enc_vq(xc)
        zq_c = _sc_gather(table, fidx_c.reshape(NH * G), NH * G, GD
                          ).reshape(NH, L)
        parts.append((z_e_c, idx_c, zq_c, loss_c))

    z_e = jnp.concatenate([p[0] for p in parts], axis=0)
    idx = jnp.concatenate([p[1] for p in parts], axis=0)
    z_q = jnp.concatenate([p[2] for p in parts], axis=0)
    loss_total = sum(p[3][0, 0] for p in parts)
    bands_hat = jnp.concatenate([dec(p[2]) for p in parts], axis=0)

    vq_loss = (2.0 * beta / (N * GD)) * loss_total
    return (bands_hat.reshape(B, T, D), z_e.reshape(B, T, L),
            z_q.reshape(B, T, L), idx.reshape(B, T, G), vq_loss)


# speedup vs baseline: 1.7160x; 1.7160x over previous
import functools

import jax
import jax.numpy as jnp
from jax.experimental import pallas as pl
from jax.experimental.pallas import tpu as pltpu


def _fused_body(x_ref, w1_ref, b1_ref, w2_ref, b2_ref, cb_ref,
                dw1_ref, db1_ref, dw2_ref, db2_ref,
                bh_ref, ze_ref, zq_ref, idx_ref, loss_ref,
                cb2_ref, cbn_ref, aug_ref,
                *, G, K, GD):
    @pl.when(pl.program_id(0) == 0)
    def _prep():
        cb = cb_ref[...]
        cb2_ref[...] = cb * 2.0
        cbn_ref[...] = jnp.sum(cb * cb, axis=1)
        aug_ref[:, :GD, :] = cb
        aug_ref[:, GD:, :] = jax.lax.broadcasted_iota(
            jnp.int32, (G, 1, K), 2).astype(jnp.float32)

    x = x_ref[...]

    h = jnp.maximum(jnp.dot(x, w1_ref[...]) + b1_ref[...], 0.0)
    z_e = jnp.dot(h, w2_ref[...]) + b2_ref[...]
    ze_ref[...] = z_e

    zq_parts = []
    idx_parts = []
    for g in range(G):
        zg = z_e[:, g * GD:(g + 1) * GD]
        s2 = jax.lax.dot_general(zg, cb2_ref[g], (((1,), (0,)), ((), ())))
        a = s2 - cbn_ref[g][None, :]
        m = jnp.max(a, axis=1, keepdims=True)
        maskf = jnp.where(a == m, 1.0, 0.0)
        r = jax.lax.dot_general(maskf, aug_ref[g],
                                (((1,), (1,)), ((), ())))
        zq_parts.append(r[:, :GD])
        idx_parts.append(r[:, GD:])

    z_q = jnp.concatenate(zq_parts, axis=1)
    zq_ref[...] = z_q
    idx_ref[...] = jnp.concatenate(idx_parts, axis=1).astype(jnp.int32)
    dz = z_q - z_e
    block_sq = jnp.sum(dz * dz)

    @pl.when(pl.program_id(0) == 0)
    def _init():
        loss_ref[...] = jnp.zeros((1, 1), jnp.float32)

    loss_ref[...] += block_sq[None, None]

    f32 = jnp.float32
    hd = jnp.maximum(
        jax.lax.dot_general(z_q.astype(jnp.bfloat16),
                            dw1_ref[...].astype(jnp.bfloat16),
                            (((1,), (0,)), ((), ())),
                            preferred_element_type=f32) + db1_ref[...], 0.0)
    bh_ref[...] = jax.lax.dot_general(
        hd.astype(jnp.bfloat16), dw2_ref[...].astype(jnp.bfloat16),
        (((1,), (0,)), ((), ())), preferred_element_type=f32) + db2_ref[...]


def kernel(bands, enc_w1, enc_b1, enc_w2, enc_b2, codebooks,
           dec_w1, dec_b1, dec_w2, dec_b2):
    B, T, D = bands.shape
    N = B * T
    G, K, GD = codebooks.shape
    H = enc_w1.shape[1]
    L = enc_w2.shape[1]
    beta = 0.25

    R = min(2048, N)
    assert N % R == 0
    grid = (N // R,)

    x = bands.reshape(N, D)
    b1 = enc_b1.reshape(1, H)
    b2 = enc_b2.reshape(1, L)
    db1 = dec_b1.reshape(1, H)
    db2 = dec_b2.reshape(1, D)

    row_spec = lambda c: pl.BlockSpec((R, c), lambda i: (i, 0))
    full2 = lambda a, b: pl.BlockSpec((a, b), lambda i: (0, 0))

    out_shapes = (
        jax.ShapeDtypeStruct((N, D), jnp.float32),
        jax.ShapeDtypeStruct((N, L), jnp.float32),
        jax.ShapeDtypeStruct((N, L), jnp.float32),
        jax.ShapeDtypeStruct((N, G), jnp.int32),
        jax.ShapeDtypeStruct((1, 1), jnp.float32),
    )

    bands_hat, z_e, z_q, idx, loss_sum = pl.pallas_call(
        functools.partial(_fused_body, G=G, K=K, GD=GD),
        grid=grid,
        in_specs=[
            row_spec(D),
            full2(D, H), full2(1, H), full2(H, L), full2(1, L),
            pl.BlockSpec((G, GD, K), lambda i: (0, 0, 0)),
            full2(L, H), full2(1, H), full2(H, D), full2(1, D),
        ],
        out_specs=(
            row_spec(D), row_spec(L), row_spec(L),
            pl.BlockSpec((R, G), lambda i: (i, 0)),
            pl.BlockSpec((1, 1), lambda i: (0, 0)),
        ),
        out_shape=out_shapes,
        scratch_shapes=[
            pltpu.VMEM((G, GD, K), jnp.float32),
            pltpu.VMEM((G, K), jnp.float32),
            pltpu.VMEM((G, GD + 1, K), jnp.float32),
        ],
    )(x, enc_w1, b1, enc_w2, b2, codebooks.transpose(0, 2, 1),
      dec_w1, db1, dec_w2, db2)

    vq_loss = (2.0 * beta / (N * GD)) * loss_sum[0, 0]
    return (bands_hat.reshape(B, T, D), z_e.reshape(B, T, L),
            z_q.reshape(B, T, L), idx.reshape(B, T, G), vq_loss)


# speedup vs baseline: 1.7173x; 1.0008x over previous
import functools

import jax
import jax.numpy as jnp
from jax.experimental import pallas as pl
from jax.experimental.pallas import tpu as pltpu


def _fused_body(x_ref, w1_ref, b1_ref, w2_ref, b2_ref, cb_ref,
                dw1_ref, db1_ref, dw2_ref, db2_ref,
                bh_ref, ze_ref, zq_ref, idx_ref, loss_ref,
                cb2_ref, cbn_ref, aug_ref,
                *, G, K, GD):
    @pl.when(pl.program_id(0) == 0)
    def _prep():
        cb = cb_ref[...]
        cb2_ref[...] = cb * 2.0
        cbn_ref[...] = jnp.sum(cb * cb, axis=1)
        aug_ref[:, :GD, :] = cb
        aug_ref[:, GD:, :] = jax.lax.broadcasted_iota(
            jnp.int32, (G, 1, K), 2).astype(jnp.float32)

    x = x_ref[...]

    h = jnp.maximum(jnp.dot(x, w1_ref[...]) + b1_ref[...], 0.0)
    z_e = jnp.dot(h, w2_ref[...]) + b2_ref[...]
    ze_ref[...] = z_e

    zq_parts = []
    idx_parts = []
    for g in range(G):
        zg = z_e[:, g * GD:(g + 1) * GD]
        s2 = jax.lax.dot_general(zg, cb2_ref[g], (((1,), (0,)), ((), ())))
        a = s2 - cbn_ref[g][None, :]
        m = jnp.max(a, axis=1, keepdims=True)
        maskf = jnp.where(a == m, 1.0, 0.0)
        r = jax.lax.dot_general(maskf, aug_ref[g],
                                (((1,), (1,)), ((), ())))
        zq_parts.append(r[:, :GD])
        idx_parts.append(r[:, GD:])

    z_q = jnp.concatenate(zq_parts, axis=1)
    zq_ref[...] = z_q
    idx_ref[...] = jnp.concatenate(idx_parts, axis=1).astype(jnp.int32)
    dz = z_q - z_e
    block_sq = jnp.sum(dz * dz)

    @pl.when(pl.program_id(0) == 0)
    def _init():
        loss_ref[...] = jnp.zeros((1, 1), jnp.float32)

    loss_ref[...] += block_sq[None, None]

    hd = jnp.maximum(jnp.dot(z_q, dw1_ref[...]) + db1_ref[...], 0.0)
    bh_ref[...] = jnp.dot(hd, dw2_ref[...]) + db2_ref[...]


def kernel(bands, enc_w1, enc_b1, enc_w2, enc_b2, codebooks,
           dec_w1, dec_b1, dec_w2, dec_b2):
    B, T, D = bands.shape
    N = B * T
    G, K, GD = codebooks.shape
    H = enc_w1.shape[1]
    L = enc_w2.shape[1]
    beta = 0.25

    R = min(2048, N)
    assert N % R == 0
    grid = (N // R,)

    x = bands.reshape(N, D)
    b1 = enc_b1.reshape(1, H)
    b2 = enc_b2.reshape(1, L)
    db1 = dec_b1.reshape(1, H)
    db2 = dec_b2.reshape(1, D)

    row_spec = lambda c: pl.BlockSpec((R, c), lambda i: (i, 0))
    full2 = lambda a, b: pl.BlockSpec((a, b), lambda i: (0, 0))

    out_shapes = (
        jax.ShapeDtypeStruct((N, D), jnp.float32),
        jax.ShapeDtypeStruct((N, L), jnp.float32),
        jax.ShapeDtypeStruct((N, L), jnp.float32),
        jax.ShapeDtypeStruct((N, G), jnp.int32),
        jax.ShapeDtypeStruct((1, 1), jnp.float32),
    )

    bands_hat, z_e, z_q, idx, loss_sum = pl.pallas_call(
        functools.partial(_fused_body, G=G, K=K, GD=GD),
        grid=grid,
        in_specs=[
            row_spec(D),
            full2(D, H), full2(1, H), full2(H, L), full2(1, L),
            pl.BlockSpec((G, GD, K), lambda i: (0, 0, 0)),
            full2(L, H), full2(1, H), full2(H, D), full2(1, D),
        ],
        out_specs=(
            row_spec(D), row_spec(L), row_spec(L),
            pl.BlockSpec((R, G), lambda i: (i, 0)),
            pl.BlockSpec((1, 1), lambda i: (0, 0)),
        ),
        out_shape=out_shapes,
        scratch_shapes=[
            pltpu.VMEM((G, GD, K), jnp.float32),
            pltpu.VMEM((G, K), jnp.float32),
            pltpu.VMEM((G, GD + 1, K), jnp.float32),
        ],
    )(x, enc_w1, b1, enc_w2, b2, codebooks.transpose(0, 2, 1),
      dec_w1, db1, dec_w2, db2)

    vq_loss = (2.0 * beta / (N * GD)) * loss_sum[0, 0]
    return (bands_hat.reshape(B, T, D), z_e.reshape(B, T, L),
            z_q.reshape(B, T, L), idx.reshape(B, T, G), vq_loss)
